# Initial kernel scaffold; baseline (speedup 1.0000x reference)
import jax
import jax.numpy as jnp
from jax.experimental import pallas as pl


def kernel(x, edge_index, batch, W1, b1, g1, be1, W2, b2, g2, be2, W3, b3, g3, be3, fcW, fcb):
    raise NotImplementedError("write your pallas kernel here")


# speedup vs baseline: 11.9372x; 11.9372x over previous
import functools

import jax
import jax.numpy as jnp
from jax import lax
from jax.experimental import pallas as pl
from jax.experimental.pallas import tpu as pltpu
from jax.experimental.pallas import tpu_sc as plsc

N = 10000
E = 320000
G = 128

NW = 32
K = 128
NB = 79
EPAD = NW * NB * K
NPAD = 10240
RPS = NPAD // 16

_MESH = plsc.VectorSubcoreMesh(core_axis_name="c", subcore_axis_name="s")
_SC_PARAMS = pltpu.CompilerParams(use_tc_tiling_on_sc=False)


def _make_edge_scatter(h_dim):
    @functools.partial(
        pl.kernel,
        mesh=_MESH,
        compiler_params=_SC_PARAMS,
        out_type=jax.ShapeDtypeStruct((2, NPAD, h_dim), jnp.float32),
        scratch_types=[
            pltpu.VMEM((NB, K), jnp.int32),
            pltpu.VMEM((NB, K), jnp.int32),
            pltpu.VMEM((K, h_dim), jnp.float32),
            pltpu.VMEM_SHARED((NPAD, h_dim), jnp.float32),
        ],
    )
    def scatter_kernel(q_hbm, src_hbm, dst_hbm, z_hbm, out_hbm,
                       src_v, dst_v, rows_v, acc):
        c = lax.axis_index("c")
        s = lax.axis_index("s")
        wid = c * 16 + s
        pltpu.sync_copy(z_hbm, acc.at[pl.ds(s * RPS, RPS)])
        pltpu.sync_copy(src_hbm.at[wid], src_v)
        pltpu.sync_copy(dst_hbm.at[wid], dst_v)
        plsc.subcore_barrier()

        @pl.loop(0, NB)
        def _(j):
            pltpu.sync_copy(q_hbm.at[src_v.at[j]], rows_v)
            pltpu.sync_copy(rows_v, acc.at[dst_v.at[j]], add=True)

        plsc.subcore_barrier()
        pltpu.sync_copy(acc.at[pl.ds(s * RPS, RPS)],
                        out_hbm.at[c, pl.ds(s * RPS, RPS)])

    return scatter_kernel


@functools.partial(
    pl.kernel,
    mesh=_MESH,
    compiler_params=_SC_PARAMS,
    out_type=jax.ShapeDtypeStruct((2, NPAD, 16), jnp.float32),
    scratch_types=[
        pltpu.VMEM((NB, K), jnp.int32),
        pltpu.VMEM((K, 16), jnp.float32),
        pltpu.VMEM_SHARED((NPAD, 16), jnp.float32),
    ],
)
def _deg_kernel(dst_hbm, ones_hbm, z_hbm, out_hbm, dst_v, ones_v, acc):
    c = lax.axis_index("c")
    s = lax.axis_index("s")
    wid = c * 16 + s
    pltpu.sync_copy(z_hbm, acc.at[pl.ds(s * RPS, RPS)])
    pltpu.sync_copy(ones_hbm, ones_v)
    pltpu.sync_copy(dst_hbm.at[wid], dst_v)
    plsc.subcore_barrier()

    @pl.loop(0, NB)
    def _(j):
        pltpu.sync_copy(ones_v, acc.at[dst_v.at[j]], add=True)

    plsc.subcore_barrier()
    pltpu.sync_copy(acc.at[pl.ds(s * RPS, RPS)],
                    out_hbm.at[c, pl.ds(s * RPS, RPS)])


_scatter128 = _make_edge_scatter(128)
_scatter64 = _make_edge_scatter(64)


def _pre_body(x_ref, w_ref, degp_ref, q_ref, dinv_ref):
    deg = degp_ref[0, :N, 0:1] + degp_ref[1, :N, 0:1] + 1.0
    dinv = lax.rsqrt(deg)
    dinv_ref[...] = dinv
    q_ref[...] = jnp.dot(x_ref[...], w_ref[...],
                         preferred_element_type=jnp.float32) * dinv


def _mid_body(sp_ref, q_ref, dinv_ref, b_ref, g_ref, be_ref, w_ref, qn_ref):
    dinv = dinv_ref[...]
    s = sp_ref[0, :N, :] + sp_ref[1, :N, :]
    t = (s + q_ref[...]) * dinv + b_ref[...]
    m = jnp.mean(t, axis=0, keepdims=True)
    cdev = t - m
    v = jnp.mean(cdev * cdev, axis=0, keepdims=True)
    h = jnp.maximum(cdev * lax.rsqrt(v + 1e-5) * g_ref[...] + be_ref[...], 0.0)
    qn_ref[...] = jnp.dot(h, w_ref[...],
                          preferred_element_type=jnp.float32) * dinv


def _post_body(sp_ref, q_ref, dinv_ref, b_ref, g_ref, be_ref, batch_ref,
               fcw_ref, fcb_ref, out_ref):
    s = sp_ref[0, :N, :] + sp_ref[1, :N, :]
    t = (s + q_ref[...]) * dinv_ref[...] + b_ref[...]
    m = jnp.mean(t, axis=0, keepdims=True)
    cdev = t - m
    v = jnp.mean(cdev * cdev, axis=0, keepdims=True)
    h = jnp.maximum(cdev * lax.rsqrt(v + 1e-5) * g_ref[...] + be_ref[...], 0.0)
    onehot_t = (batch_ref[...] ==
                lax.broadcasted_iota(jnp.int32, (G, N), 0)).astype(jnp.float32)
    sums = jnp.dot(onehot_t, h, preferred_element_type=jnp.float32)
    cnt = jnp.sum(onehot_t, axis=1, keepdims=True)
    pooled = sums / jnp.maximum(cnt, 1.0)
    out_ref[...] = jnp.dot(pooled, fcw_ref[...],
                           preferred_element_type=jnp.float32) + fcb_ref[...]


def kernel(x, edge_index, batch, W1, b1, g1, be1, W2, b2, g2, be2,
           W3, b3, g3, be3, fcW, fcb):
    f32 = jnp.float32
    pad = EPAD - E
    srcp = jnp.concatenate(
        [edge_index[0], jnp.zeros((pad,), jnp.int32)]).reshape(NW, NB, K)
    dstp = jnp.concatenate(
        [edge_index[1], jnp.full((pad,), N, jnp.int32)]).reshape(NW, NB, K)
    z16 = jnp.zeros((RPS, 16), f32)
    z64 = jnp.zeros((RPS, 64), f32)
    z128 = jnp.zeros((RPS, 128), f32)
    ones16 = jnp.ones((K, 16), f32)

    degp = _deg_kernel(dstp, ones16, z16)

    q1, dinv = pl.pallas_call(
        _pre_body,
        out_shape=[jax.ShapeDtypeStruct((N, 128), f32),
                   jax.ShapeDtypeStruct((N, 1), f32)],
    )(x, W1, degp)

    s1 = _scatter128(q1, srcp, dstp, z128)
    q2 = pl.pallas_call(
        _mid_body, out_shape=jax.ShapeDtypeStruct((N, 128), f32),
    )(s1, q1, dinv, b1.reshape(1, -1), g1.reshape(1, -1), be1.reshape(1, -1),
      W2)

    s2 = _scatter128(q2, srcp, dstp, z128)
    q3 = pl.pallas_call(
        _mid_body, out_shape=jax.ShapeDtypeStruct((N, 64), f32),
    )(s2, q2, dinv, b2.reshape(1, -1), g2.reshape(1, -1), be2.reshape(1, -1),
      W3)

    s3 = _scatter64(q3, srcp, dstp, z64)
    out = pl.pallas_call(
        _post_body, out_shape=jax.ShapeDtypeStruct((G, 10), f32),
    )(s3, q3, dinv, b3.reshape(1, -1), g3.reshape(1, -1), be3.reshape(1, -1),
      batch.reshape(1, -1), fcW, fcb.reshape(1, -1))
    return out


# speedup vs baseline: 14.2318x; 1.1922x over previous
import functools

import jax
import jax.numpy as jnp
from jax import lax
from jax.experimental import pallas as pl
from jax.experimental.pallas import tpu as pltpu
from jax.experimental.pallas import tpu_sc as plsc

N = 10000
E = 320000
G = 128

K = 128
NBT = 160
EPAD = 16 * NBT * K
NPAD = 10240
RPS = NPAD // 16

_MESH = plsc.VectorSubcoreMesh(core_axis_name="c", subcore_axis_name="s")
_SC_PARAMS = pltpu.CompilerParams(use_tc_tiling_on_sc=False)


def _make_edge_scatter(hh):
    @functools.partial(
        pl.kernel,
        mesh=_MESH,
        compiler_params=_SC_PARAMS,
        out_type=jax.ShapeDtypeStruct((2, NPAD, hh), jnp.float32),
        scratch_types=[
            pltpu.VMEM((NBT, K), jnp.int32),
            pltpu.VMEM((NBT, K), jnp.int32),
            pltpu.VMEM((K, hh), jnp.float32),
            pltpu.VMEM((K, hh), jnp.float32),
            pltpu.VMEM_SHARED((NPAD, hh), jnp.float32),
            pltpu.SemaphoreType.DMA,
            pltpu.SemaphoreType.DMA,
        ],
    )
    def scatter_kernel(q_hbm, src_hbm, dst_hbm, z_hbm, out_hbm,
                       src_v, dst_v, buf_a, buf_b, acc, sem_a, sem_b):
        c = lax.axis_index("c")
        s = lax.axis_index("s")
        pltpu.sync_copy(z_hbm, acc.at[pl.ds(s * RPS, RPS)])
        pltpu.sync_copy(src_hbm.at[s], src_v)
        pltpu.sync_copy(dst_hbm.at[s], dst_v)
        plsc.subcore_barrier()

        pltpu.async_copy(q_hbm.at[c].at[src_v.at[0]], buf_a, sem_a)
        pltpu.async_copy(q_hbm.at[c].at[src_v.at[1]], buf_b, sem_b)

        @pl.loop(0, NBT, step=2)
        def _(j):
            pltpu.make_async_copy(
                q_hbm.at[c].at[src_v.at[j]], buf_a, sem_a).wait()
            pltpu.sync_copy(buf_a, acc.at[dst_v.at[j]], add=True)

            @pl.when(j + 2 < NBT)
            def _():
                pltpu.async_copy(q_hbm.at[c].at[src_v.at[j + 2]], buf_a, sem_a)

            pltpu.make_async_copy(
                q_hbm.at[c].at[src_v.at[j + 1]], buf_b, sem_b).wait()
            pltpu.sync_copy(buf_b, acc.at[dst_v.at[j + 1]], add=True)

            @pl.when(j + 3 < NBT)
            def _():
                pltpu.async_copy(q_hbm.at[c].at[src_v.at[j + 3]], buf_b, sem_b)

        plsc.subcore_barrier()
        pltpu.sync_copy(acc.at[pl.ds(s * RPS, RPS)],
                        out_hbm.at[c, pl.ds(s * RPS, RPS)])

    return scatter_kernel


@functools.partial(
    pl.kernel,
    mesh=_MESH,
    compiler_params=_SC_PARAMS,
    out_type=jax.ShapeDtypeStruct((2, NPAD, 16), jnp.float32),
    scratch_types=[
        pltpu.VMEM((NBT // 2, K), jnp.int32),
        pltpu.VMEM((K, 16), jnp.float32),
        pltpu.VMEM_SHARED((NPAD, 16), jnp.float32),
    ],
)
def _deg_kernel(dst_hbm, ones_hbm, z_hbm, out_hbm, dst_v, ones_v, acc):
    c = lax.axis_index("c")
    s = lax.axis_index("s")
    wid = c * 16 + s
    pltpu.sync_copy(z_hbm, acc.at[pl.ds(s * RPS, RPS)])
    pltpu.sync_copy(ones_hbm, ones_v)
    pltpu.sync_copy(dst_hbm.at[wid], dst_v)
    plsc.subcore_barrier()

    @pl.loop(0, NBT // 2)
    def _(j):
        pltpu.sync_copy(ones_v, acc.at[dst_v.at[j]], add=True)

    plsc.subcore_barrier()
    pltpu.sync_copy(acc.at[pl.ds(s * RPS, RPS)],
                    out_hbm.at[c, pl.ds(s * RPS, RPS)])


_scatter64 = _make_edge_scatter(64)
_scatter32 = _make_edge_scatter(32)


def _pre_body(x_ref, w_ref, degp_ref, q_ref, dinv_ref):
    deg = degp_ref[0, :N, 0:1] + degp_ref[1, :N, 0:1] + 1.0
    dinv = lax.rsqrt(deg)
    dinv_ref[...] = dinv
    q = jnp.dot(x_ref[...], w_ref[...],
                preferred_element_type=jnp.float32) * dinv
    q_ref[0, :, :] = q[:, :64]
    q_ref[1, :, :] = q[:, 64:]


def _mid_body(sp_ref, q_ref, dinv_ref, b_ref, g_ref, be_ref, w_ref, qn_ref):
    hh = qn_ref.shape[2]
    dinv = dinv_ref[...]
    s = jnp.concatenate([sp_ref[0, :N, :], sp_ref[1, :N, :]], axis=1)
    q = jnp.concatenate([q_ref[0], q_ref[1]], axis=1)
    t = (s + q) * dinv + b_ref[...]
    m = jnp.mean(t, axis=0, keepdims=True)
    cdev = t - m
    v = jnp.mean(cdev * cdev, axis=0, keepdims=True)
    h = jnp.maximum(cdev * lax.rsqrt(v + 1e-5) * g_ref[...] + be_ref[...], 0.0)
    qn = jnp.dot(h, w_ref[...], preferred_element_type=jnp.float32) * dinv
    qn_ref[0, :, :] = qn[:, :hh]
    qn_ref[1, :, :] = qn[:, hh:]


def _post_body(sp_ref, q_ref, dinv_ref, b_ref, g_ref, be_ref, batch_ref,
               fcw_ref, fcb_ref, out_ref):
    s = jnp.concatenate([sp_ref[0, :N, :], sp_ref[1, :N, :]], axis=1)
    q = jnp.concatenate([q_ref[0], q_ref[1]], axis=1)
    t = (s + q) * dinv_ref[...] + b_ref[...]
    m = jnp.mean(t, axis=0, keepdims=True)
    cdev = t - m
    v = jnp.mean(cdev * cdev, axis=0, keepdims=True)
    h = jnp.maximum(cdev * lax.rsqrt(v + 1e-5) * g_ref[...] + be_ref[...], 0.0)
    onehot_t = (batch_ref[...] ==
                lax.broadcasted_iota(jnp.int32, (G, N), 0)).astype(jnp.float32)
    sums = jnp.dot(onehot_t, h, preferred_element_type=jnp.float32)
    cnt = jnp.sum(onehot_t, axis=1, keepdims=True)
    pooled = sums / jnp.maximum(cnt, 1.0)
    out_ref[...] = jnp.dot(pooled, fcw_ref[...],
                           preferred_element_type=jnp.float32) + fcb_ref[...]


def kernel(x, edge_index, batch, W1, b1, g1, be1, W2, b2, g2, be2,
           W3, b3, g3, be3, fcW, fcb):
    f32 = jnp.float32
    pad = EPAD - E
    srcp = jnp.concatenate(
        [edge_index[0], jnp.zeros((pad,), jnp.int32)]).reshape(16, NBT, K)
    dstp = jnp.concatenate(
        [edge_index[1], jnp.full((pad,), N, jnp.int32)]).reshape(16, NBT, K)
    dstp32 = dstp.reshape(32, NBT // 2, K)
    z16 = jnp.zeros((RPS, 16), f32)
    z32 = jnp.zeros((RPS, 32), f32)
    z64 = jnp.zeros((RPS, 64), f32)
    ones16 = jnp.ones((K, 16), f32)

    degp = _deg_kernel(dstp32, ones16, z16)

    q1, dinv = pl.pallas_call(
        _pre_body,
        out_shape=[jax.ShapeDtypeStruct((2, N, 64), f32),
                   jax.ShapeDtypeStruct((N, 1), f32)],
    )(x, W1, degp)

    s1 = _scatter64(q1, srcp, dstp, z64)
    q2 = pl.pallas_call(
        _mid_body, out_shape=jax.ShapeDtypeStruct((2, N, 64), f32),
    )(s1, q1, dinv, b1.reshape(1, -1), g1.reshape(1, -1), be1.reshape(1, -1),
      W2)

    s2 = _scatter64(q2, srcp, dstp, z64)
    q3 = pl.pallas_call(
        _mid_body, out_shape=jax.ShapeDtypeStruct((2, N, 32), f32),
    )(s2, q2, dinv, b2.reshape(1, -1), g2.reshape(1, -1), be2.reshape(1, -1),
      W3)

    s3 = _scatter32(q3, srcp, dstp, z32)
    out = pl.pallas_call(
        _post_body, out_shape=jax.ShapeDtypeStruct((G, 10), f32),
    )(s3, q3, dinv, b3.reshape(1, -1), g3.reshape(1, -1), be3.reshape(1, -1),
      batch.reshape(1, -1), fcW, fcb.reshape(1, -1))
    return out


# speedup vs baseline: 14.4234x; 1.0135x over previous
import functools

import jax
import jax.numpy as jnp
from jax import lax
from jax.experimental import pallas as pl
from jax.experimental.pallas import tpu as pltpu
from jax.experimental.pallas import tpu_sc as plsc

N = 10000
E = 320000
G = 128

K = 128
NBT = 160
EPAD = 16 * NBT * K
NPAD = 10240
RPS = NPAD // 16

_MESH = plsc.VectorSubcoreMesh(core_axis_name="c", subcore_axis_name="s")
_SC_PARAMS = pltpu.CompilerParams(use_tc_tiling_on_sc=False)


def _make_edge_scatter(hh):
    @functools.partial(
        pl.kernel,
        mesh=_MESH,
        compiler_params=_SC_PARAMS,
        out_type=jax.ShapeDtypeStruct((2, NPAD, hh), jnp.float32),
        scratch_types=[
            pltpu.VMEM((NBT, K), jnp.int32),
            pltpu.VMEM((NBT, K), jnp.int32),
            pltpu.VMEM((K, hh), jnp.float32),
            pltpu.VMEM((K, hh), jnp.float32),
            pltpu.VMEM((K, hh), jnp.float32),
            pltpu.VMEM((K, hh), jnp.float32),
            pltpu.VMEM_SHARED((NPAD, hh), jnp.float32),
            pltpu.SemaphoreType.DMA,
            pltpu.SemaphoreType.DMA,
            pltpu.SemaphoreType.DMA,
            pltpu.SemaphoreType.DMA,
            pltpu.SemaphoreType.DMA,
            pltpu.SemaphoreType.DMA,
            pltpu.SemaphoreType.DMA,
            pltpu.SemaphoreType.DMA,
        ],
    )
    def scatter_kernel(q_hbm, src_hbm, dst_hbm, z_hbm, out_hbm,
                       src_v, dst_v, b0, b1, b2, b3, acc,
                       g0, g1, g2, g3, s0, s1, s2, s3):
        c = lax.axis_index("c")
        s = lax.axis_index("s")
        bufs = [b0, b1, b2, b3]
        gsems = [g0, g1, g2, g3]
        ssems = [s0, s1, s2, s3]
        pltpu.sync_copy(z_hbm, acc.at[pl.ds(s * RPS, RPS)])
        pltpu.sync_copy(src_hbm.at[s], src_v)
        pltpu.sync_copy(dst_hbm.at[s], dst_v)
        plsc.subcore_barrier()

        def g_issue(j, b):
            pltpu.async_copy(q_hbm.at[c].at[src_v.at[j]], bufs[b], gsems[b])

        def g_wait(j, b):
            pltpu.make_async_copy(
                q_hbm.at[c].at[src_v.at[j]], bufs[b], gsems[b]).wait()

        def s_issue(j, b):
            pltpu.async_copy(bufs[b], acc.at[dst_v.at[j]], ssems[b], add=True)

        def s_wait(j, b):
            pltpu.make_async_copy(
                bufs[b], acc.at[dst_v.at[j]], ssems[b]).wait()

        g_issue(0, 0)
        g_issue(1, 1)
        g_wait(0, 0); s_issue(0, 0); g_issue(2, 2)
        g_wait(1, 1); s_issue(1, 1); g_issue(3, 3)
        g_wait(2, 2); s_issue(2, 2); s_wait(0, 0); g_issue(4, 0)
        g_wait(3, 3); s_issue(3, 3); s_wait(1, 1); g_issue(5, 1)

        @pl.loop(4, NBT - 4, step=4)
        def _(jr):
            for b in range(4):
                j = jr + b
                b2 = (b + 2) % 4
                g_wait(j, b)
                s_issue(j, b)
                s_wait(j - 2, b2)
                g_issue(j + 2, b2)

        jr = NBT - 4
        g_wait(jr, 0); s_issue(jr, 0); s_wait(jr - 2, 2); g_issue(jr + 2, 2)
        g_wait(jr + 1, 1); s_issue(jr + 1, 1); s_wait(jr - 1, 3)
        g_issue(jr + 3, 3)
        g_wait(jr + 2, 2); s_issue(jr + 2, 2); s_wait(jr, 0)
        g_wait(jr + 3, 3); s_issue(jr + 3, 3); s_wait(jr + 1, 1)
        s_wait(jr + 2, 2)
        s_wait(jr + 3, 3)

        plsc.subcore_barrier()
        pltpu.sync_copy(acc.at[pl.ds(s * RPS, RPS)],
                        out_hbm.at[c, pl.ds(s * RPS, RPS)])

    return scatter_kernel


@functools.partial(
    pl.kernel,
    mesh=_MESH,
    compiler_params=_SC_PARAMS,
    out_type=jax.ShapeDtypeStruct((2, NPAD, 16), jnp.float32),
    scratch_types=[
        pltpu.VMEM((NBT // 2, K), jnp.int32),
        pltpu.VMEM((K, 16), jnp.float32),
        pltpu.VMEM_SHARED((NPAD, 16), jnp.float32),
    ],
)
def _deg_kernel(dst_hbm, ones_hbm, z_hbm, out_hbm, dst_v, ones_v, acc):
    c = lax.axis_index("c")
    s = lax.axis_index("s")
    wid = c * 16 + s
    pltpu.sync_copy(z_hbm, acc.at[pl.ds(s * RPS, RPS)])
    pltpu.sync_copy(ones_hbm, ones_v)
    pltpu.sync_copy(dst_hbm.at[wid], dst_v)
    plsc.subcore_barrier()

    @pl.loop(0, NBT // 2)
    def _(j):
        pltpu.sync_copy(ones_v, acc.at[dst_v.at[j]], add=True)

    plsc.subcore_barrier()
    pltpu.sync_copy(acc.at[pl.ds(s * RPS, RPS)],
                    out_hbm.at[c, pl.ds(s * RPS, RPS)])


_scatter64 = _make_edge_scatter(64)
_scatter32 = _make_edge_scatter(32)


def _pre_body(x_ref, w_ref, degp_ref, q_ref, dinv_ref):
    deg = degp_ref[0, :N, 0:1] + degp_ref[1, :N, 0:1] + 1.0
    dinv = lax.rsqrt(deg)
    dinv_ref[...] = dinv
    q = jnp.dot(x_ref[...], w_ref[...],
                preferred_element_type=jnp.float32) * dinv
    q_ref[0, :, :] = q[:, :64]
    q_ref[1, :, :] = q[:, 64:]


def _mid_body(sp_ref, q_ref, dinv_ref, b_ref, g_ref, be_ref, w_ref, qn_ref):
    hh = qn_ref.shape[2]
    dinv = dinv_ref[...]
    s = jnp.concatenate([sp_ref[0, :N, :], sp_ref[1, :N, :]], axis=1)
    q = jnp.concatenate([q_ref[0], q_ref[1]], axis=1)
    t = (s + q) * dinv + b_ref[...]
    m = jnp.mean(t, axis=0, keepdims=True)
    cdev = t - m
    v = jnp.mean(cdev * cdev, axis=0, keepdims=True)
    h = jnp.maximum(cdev * lax.rsqrt(v + 1e-5) * g_ref[...] + be_ref[...], 0.0)
    qn = jnp.dot(h, w_ref[...], preferred_element_type=jnp.float32) * dinv
    qn_ref[0, :, :] = qn[:, :hh]
    qn_ref[1, :, :] = qn[:, hh:]


def _post_body(sp_ref, q_ref, dinv_ref, b_ref, g_ref, be_ref, batch_ref,
               fcw_ref, fcb_ref, out_ref):
    s = jnp.concatenate([sp_ref[0, :N, :], sp_ref[1, :N, :]], axis=1)
    q = jnp.concatenate([q_ref[0], q_ref[1]], axis=1)
    t = (s + q) * dinv_ref[...] + b_ref[...]
    m = jnp.mean(t, axis=0, keepdims=True)
    cdev = t - m
    v = jnp.mean(cdev * cdev, axis=0, keepdims=True)
    h = jnp.maximum(cdev * lax.rsqrt(v + 1e-5) * g_ref[...] + be_ref[...], 0.0)
    onehot_t = (batch_ref[...] ==
                lax.broadcasted_iota(jnp.int32, (G, N), 0)).astype(jnp.float32)
    sums = jnp.dot(onehot_t, h, preferred_element_type=jnp.float32)
    cnt = jnp.sum(onehot_t, axis=1, keepdims=True)
    pooled = sums / jnp.maximum(cnt, 1.0)
    out_ref[...] = jnp.dot(pooled, fcw_ref[...],
                           preferred_element_type=jnp.float32) + fcb_ref[...]


def kernel(x, edge_index, batch, W1, b1, g1, be1, W2, b2, g2, be2,
           W3, b3, g3, be3, fcW, fcb):
    f32 = jnp.float32
    pad = EPAD - E
    srcp = jnp.concatenate(
        [edge_index[0], jnp.zeros((pad,), jnp.int32)]).reshape(16, NBT, K)
    dstp = jnp.concatenate(
        [edge_index[1], jnp.full((pad,), N, jnp.int32)]).reshape(16, NBT, K)
    dstp32 = dstp.reshape(32, NBT // 2, K)
    z16 = jnp.zeros((RPS, 16), f32)
    z32 = jnp.zeros((RPS, 32), f32)
    z64 = jnp.zeros((RPS, 64), f32)
    ones16 = jnp.ones((K, 16), f32)

    degp = _deg_kernel(dstp32, ones16, z16)

    q1, dinv = pl.pallas_call(
        _pre_body,
        out_shape=[jax.ShapeDtypeStruct((2, N, 64), f32),
                   jax.ShapeDtypeStruct((N, 1), f32)],
    )(x, W1, degp)

    s1 = _scatter64(q1, srcp, dstp, z64)
    q2 = pl.pallas_call(
        _mid_body, out_shape=jax.ShapeDtypeStruct((2, N, 64), f32),
    )(s1, q1, dinv, b1.reshape(1, -1), g1.reshape(1, -1), be1.reshape(1, -1),
      W2)

    s2 = _scatter64(q2, srcp, dstp, z64)
    q3 = pl.pallas_call(
        _mid_body, out_shape=jax.ShapeDtypeStruct((2, N, 32), f32),
    )(s2, q2, dinv, b2.reshape(1, -1), g2.reshape(1, -1), be2.reshape(1, -1),
      W3)

    s3 = _scatter32(q3, srcp, dstp, z32)
    out = pl.pallas_call(
        _post_body, out_shape=jax.ShapeDtypeStruct((G, 10), f32),
    )(s3, q3, dinv, b3.reshape(1, -1), g3.reshape(1, -1), be3.reshape(1, -1),
      batch.reshape(1, -1), fcW, fcb.reshape(1, -1))
    return out


# speedup vs baseline: 22.2780x; 1.5446x over previous
import functools

import jax
import jax.numpy as jnp
from jax import lax
from jax.experimental import pallas as pl
from jax.experimental.pallas import tpu as pltpu
from jax.experimental.pallas import tpu_sc as plsc

N = 10000
E = 320000
G = 128

K = 128
NBT = 160
NBH = NBT // 2
EPAD = 16 * NBT * K
NPAD = 10016
RPS = NPAD // 16
TRS = N // 16

_MESH = plsc.VectorSubcoreMesh(core_axis_name="c", subcore_axis_name="s")
_SC_PARAMS = pltpu.CompilerParams(use_tc_tiling_on_sc=False)


def _make_edge_scatter(hh):
    @functools.partial(
        pl.kernel,
        mesh=_MESH,
        compiler_params=_SC_PARAMS,
        out_type=jax.ShapeDtypeStruct((2, NPAD, hh), jnp.float32),
        scratch_types=[
            pltpu.VMEM((NBH, K), jnp.int32),
            pltpu.VMEM((NBH, K), jnp.int32),
            pltpu.VMEM((K, hh), jnp.float32),
            pltpu.VMEM((K, hh), jnp.float32),
            pltpu.VMEM_SHARED((N, hh), jnp.float32),
            pltpu.VMEM_SHARED((NPAD, hh), jnp.float32),
            pltpu.SemaphoreType.DMA,
            pltpu.SemaphoreType.DMA,
            pltpu.SemaphoreType.DMA,
            pltpu.SemaphoreType.DMA,
        ],
    )
    def scatter_kernel(q_hbm, src_hbm, dst_hbm, z_hbm, out_hbm,
                       src_v, dst_v, b0, b1, table, acc, g0, g1, s0, s1):
        c = lax.axis_index("c")
        s = lax.axis_index("s")
        bufs = [b0, b1]
        gsems = [g0, g1]
        ssems = [s0, s1]
        pltpu.sync_copy(z_hbm, acc.at[pl.ds(s * RPS, RPS)])
        pltpu.sync_copy(q_hbm.at[c].at[pl.ds(s * TRS, TRS)],
                        table.at[pl.ds(s * TRS, TRS)])
        pltpu.sync_copy(src_hbm.at[s, pl.ds(0, NBH)], src_v)
        pltpu.sync_copy(dst_hbm.at[s, pl.ds(0, NBH)], dst_v)
        plsc.subcore_barrier()

        def g_issue(j, b):
            pltpu.async_copy(table.at[src_v.at[j]], bufs[b], gsems[b])

        def g_wait(j, b):
            pltpu.make_async_copy(
                table.at[src_v.at[j]], bufs[b], gsems[b]).wait()

        def s_issue(j, b):
            pltpu.async_copy(bufs[b], acc.at[dst_v.at[j]], ssems[b],
                             add=True)

        def s_wait(j, b):
            pltpu.make_async_copy(
                bufs[b], acc.at[dst_v.at[j]], ssems[b]).wait()

        def run_half():
            g_issue(0, 0)
            g_wait(0, 0); s_issue(0, 0); g_issue(1, 1)
            g_wait(1, 1); s_issue(1, 1); s_wait(0, 0); g_issue(2, 0)

            @pl.loop(2, NBH - 2, step=2)
            def _(jr):
                for b in range(2):
                    j = jr + b
                    g_wait(j, b)
                    s_issue(j, b)
                    s_wait(j - 1, 1 - b)
                    g_issue(j + 1, 1 - b)

            jr = NBH - 2
            g_wait(jr, 0); s_issue(jr, 0); s_wait(jr - 1, 1)
            g_issue(jr + 1, 1)
            g_wait(jr + 1, 1); s_issue(jr + 1, 1); s_wait(jr, 0)
            s_wait(jr + 1, 1)

        run_half()
        pltpu.sync_copy(src_hbm.at[s, pl.ds(NBH, NBH)], src_v)
        pltpu.sync_copy(dst_hbm.at[s, pl.ds(NBH, NBH)], dst_v)
        run_half()

        plsc.subcore_barrier()
        pltpu.sync_copy(acc.at[pl.ds(s * RPS, RPS)],
                        out_hbm.at[c, pl.ds(s * RPS, RPS)])

    return scatter_kernel


@functools.partial(
    pl.kernel,
    mesh=_MESH,
    compiler_params=_SC_PARAMS,
    out_type=jax.ShapeDtypeStruct((2, NPAD, 16), jnp.float32),
    scratch_types=[
        pltpu.VMEM((NBT // 2, K), jnp.int32),
        pltpu.VMEM((K, 16), jnp.float32),
        pltpu.VMEM_SHARED((NPAD, 16), jnp.float32),
    ],
)
def _deg_kernel(dst_hbm, ones_hbm, z_hbm, out_hbm, dst_v, ones_v, acc):
    c = lax.axis_index("c")
    s = lax.axis_index("s")
    wid = c * 16 + s
    pltpu.sync_copy(z_hbm, acc.at[pl.ds(s * RPS, RPS)])
    pltpu.sync_copy(ones_hbm, ones_v)
    pltpu.sync_copy(dst_hbm.at[wid], dst_v)
    plsc.subcore_barrier()

    @pl.loop(0, NBT // 2)
    def _(j):
        pltpu.sync_copy(ones_v, acc.at[dst_v.at[j]], add=True)

    plsc.subcore_barrier()
    pltpu.sync_copy(acc.at[pl.ds(s * RPS, RPS)],
                    out_hbm.at[c, pl.ds(s * RPS, RPS)])


_scatter64 = _make_edge_scatter(64)
_scatter32 = _make_edge_scatter(32)


def _pre_body(x_ref, w_ref, degp_ref, q_ref, dinv_ref):
    deg = degp_ref[0, :N, 0:1] + degp_ref[1, :N, 0:1] + 1.0
    dinv = lax.rsqrt(deg)
    dinv_ref[...] = dinv
    q = jnp.dot(x_ref[...], w_ref[...],
                preferred_element_type=jnp.float32) * dinv
    q_ref[0, :, :] = q[:, :64]
    q_ref[1, :, :] = q[:, 64:]


def _mid_body(sp_ref, q_ref, dinv_ref, b_ref, g_ref, be_ref, w_ref, qn_ref):
    hh = qn_ref.shape[2]
    dinv = dinv_ref[...]
    s = jnp.concatenate([sp_ref[0, :N, :], sp_ref[1, :N, :]], axis=1)
    q = jnp.concatenate([q_ref[0], q_ref[1]], axis=1)
    t = (s + q) * dinv + b_ref[...]
    m = jnp.mean(t, axis=0, keepdims=True)
    cdev = t - m
    v = jnp.mean(cdev * cdev, axis=0, keepdims=True)
    h = jnp.maximum(cdev * lax.rsqrt(v + 1e-5) * g_ref[...] + be_ref[...], 0.0)
    qn = jnp.dot(h, w_ref[...], preferred_element_type=jnp.float32) * dinv
    qn_ref[0, :, :] = qn[:, :hh]
    qn_ref[1, :, :] = qn[:, hh:]


def _post_body(sp_ref, q_ref, dinv_ref, b_ref, g_ref, be_ref, batch_ref,
               fcw_ref, fcb_ref, out_ref):
    s = jnp.concatenate([sp_ref[0, :N, :], sp_ref[1, :N, :]], axis=1)
    q = jnp.concatenate([q_ref[0], q_ref[1]], axis=1)
    t = (s + q) * dinv_ref[...] + b_ref[...]
    m = jnp.mean(t, axis=0, keepdims=True)
    cdev = t - m
    v = jnp.mean(cdev * cdev, axis=0, keepdims=True)
    h = jnp.maximum(cdev * lax.rsqrt(v + 1e-5) * g_ref[...] + be_ref[...], 0.0)
    onehot_t = (batch_ref[...] ==
                lax.broadcasted_iota(jnp.int32, (G, N), 0)).astype(jnp.float32)
    sums = jnp.dot(onehot_t, h, preferred_element_type=jnp.float32)
    cnt = jnp.sum(onehot_t, axis=1, keepdims=True)
    pooled = sums / jnp.maximum(cnt, 1.0)
    out_ref[...] = jnp.dot(pooled, fcw_ref[...],
                           preferred_element_type=jnp.float32) + fcb_ref[...]


def kernel(x, edge_index, batch, W1, b1, g1, be1, W2, b2, g2, be2,
           W3, b3, g3, be3, fcW, fcb):
    f32 = jnp.float32
    pad = EPAD - E
    srcp = jnp.concatenate(
        [edge_index[0], jnp.zeros((pad,), jnp.int32)]).reshape(16, NBT, K)
    dstp = jnp.concatenate(
        [edge_index[1], jnp.full((pad,), N, jnp.int32)]).reshape(16, NBT, K)
    dstp32 = dstp.reshape(32, NBT // 2, K)
    z16 = jnp.zeros((RPS, 16), f32)
    z32 = jnp.zeros((RPS, 32), f32)
    z64 = jnp.zeros((RPS, 64), f32)
    ones16 = jnp.ones((K, 16), f32)

    degp = _deg_kernel(dstp32, ones16, z16)

    q1, dinv = pl.pallas_call(
        _pre_body,
        out_shape=[jax.ShapeDtypeStruct((2, N, 64), f32),
                   jax.ShapeDtypeStruct((N, 1), f32)],
    )(x, W1, degp)

    s1 = _scatter64(q1, srcp, dstp, z64)
    q2 = pl.pallas_call(
        _mid_body, out_shape=jax.ShapeDtypeStruct((2, N, 64), f32),
    )(s1, q1, dinv, b1.reshape(1, -1), g1.reshape(1, -1), be1.reshape(1, -1),
      W2)

    s2 = _scatter64(q2, srcp, dstp, z64)
    q3 = pl.pallas_call(
        _mid_body, out_shape=jax.ShapeDtypeStruct((2, N, 32), f32),
    )(s2, q2, dinv, b2.reshape(1, -1), g2.reshape(1, -1), be2.reshape(1, -1),
      W3)

    s3 = _scatter32(q3, srcp, dstp, z32)
    out = pl.pallas_call(
        _post_body, out_shape=jax.ShapeDtypeStruct((G, 10), f32),
    )(s3, q3, dinv, b3.reshape(1, -1), g3.reshape(1, -1), be3.reshape(1, -1),
      batch.reshape(1, -1), fcW, fcb.reshape(1, -1))
    return out


# speedup vs baseline: 25.6590x; 1.1518x over previous
import functools

import jax
import jax.numpy as jnp
from jax import lax
from jax.experimental import pallas as pl
from jax.experimental.pallas import tpu as pltpu
from jax.experimental.pallas import tpu_sc as plsc

N = 10000
E = 320000
G = 128

K = 128
NBT = 160
NBH = NBT // 2
EPAD = 16 * NBT * K
NPAD = 10016
RPS = NPAD // 16
TRS = N // 16


_MESH = plsc.VectorSubcoreMesh(core_axis_name="c", subcore_axis_name="s")
_SC_PARAMS = pltpu.CompilerParams(use_tc_tiling_on_sc=False)


def _make_edge_scatter(hh):
    @functools.partial(
        pl.kernel,
        mesh=_MESH,
        compiler_params=_SC_PARAMS,
        out_type=jax.ShapeDtypeStruct((NPAD, 128), jnp.float32),
        scratch_types=[
            pltpu.VMEM((NBH, K), jnp.int32),
            pltpu.VMEM((NBH, K), jnp.int32),
            pltpu.VMEM((K, hh), jnp.float32),
            pltpu.VMEM((K, hh), jnp.float32),
            pltpu.VMEM_SHARED((N, hh), jnp.float32),
            pltpu.VMEM_SHARED((NPAD, hh), jnp.float32),
            pltpu.SemaphoreType.DMA,
            pltpu.SemaphoreType.DMA,
            pltpu.SemaphoreType.DMA,
            pltpu.SemaphoreType.DMA,
        ],
    )
    def scatter_kernel(q_hbm, src_hbm, dst_hbm, z_hbm, out_hbm,
                       src_v, dst_v, b0, b1, table, acc, g0, g1, s0, s1):
        c = lax.axis_index("c")
        s = lax.axis_index("s")
        bufs = [b0, b1]
        gsems = [g0, g1]
        ssems = [s0, s1]
        pltpu.sync_copy(z_hbm.at[:, pl.ds(0, hh)],
                        acc.at[pl.ds(s * RPS, RPS)])
        pltpu.sync_copy(q_hbm.at[pl.ds(s * TRS, TRS), pl.ds(c * hh, hh)],
                        table.at[pl.ds(s * TRS, TRS)])
        pltpu.sync_copy(src_hbm.at[s, pl.ds(0, NBH)], src_v)
        pltpu.sync_copy(dst_hbm.at[s, pl.ds(0, NBH)], dst_v)
        plsc.subcore_barrier()

        def g_issue(j, b):
            pltpu.async_copy(table.at[src_v.at[j]], bufs[b], gsems[b])

        def g_wait(j, b):
            pltpu.make_async_copy(
                table.at[src_v.at[j]], bufs[b], gsems[b]).wait()

        def s_issue(j, b):
            pltpu.async_copy(bufs[b], acc.at[dst_v.at[j]], ssems[b],
                             add=True)

        def s_wait(j, b):
            pltpu.make_async_copy(
                bufs[b], acc.at[dst_v.at[j]], ssems[b]).wait()

        def run_half():
            g_issue(0, 0)
            g_wait(0, 0); s_issue(0, 0); g_issue(1, 1)
            g_wait(1, 1); s_issue(1, 1); s_wait(0, 0); g_issue(2, 0)

            @pl.loop(2, NBH - 2, step=2)
            def _(jr):
                for b in range(2):
                    j = jr + b
                    g_wait(j, b)
                    s_issue(j, b)
                    s_wait(j - 1, 1 - b)
                    g_issue(j + 1, 1 - b)

            jr = NBH - 2
            g_wait(jr, 0); s_issue(jr, 0); s_wait(jr - 1, 1)
            g_issue(jr + 1, 1)
            g_wait(jr + 1, 1); s_issue(jr + 1, 1); s_wait(jr, 0)
            s_wait(jr + 1, 1)

        run_half()
        pltpu.sync_copy(src_hbm.at[s, pl.ds(NBH, NBH)], src_v)
        pltpu.sync_copy(dst_hbm.at[s, pl.ds(NBH, NBH)], dst_v)
        run_half()

        plsc.subcore_barrier()
        pltpu.sync_copy(acc.at[pl.ds(s * RPS, RPS)],
                        out_hbm.at[pl.ds(s * RPS, RPS), pl.ds(c * hh, hh)])

    return scatter_kernel


@functools.partial(
    pl.kernel,
    mesh=_MESH,
    compiler_params=_SC_PARAMS,
    out_type=jax.ShapeDtypeStruct((NPAD, 128), jnp.float32),
    scratch_types=[
        pltpu.VMEM((NBH, K), jnp.int32),
        pltpu.VMEM((K, 16), jnp.float32),
        pltpu.VMEM_SHARED((NPAD, 16), jnp.float32),
    ],
)
def _deg_kernel(dst_hbm, ones_hbm, z_hbm, out_hbm, dst_v, ones_v, acc):
    c = lax.axis_index("c")
    s = lax.axis_index("s")
    wid = c * 16 + s
    pltpu.sync_copy(z_hbm.at[:, pl.ds(0, 16)], acc.at[pl.ds(s * RPS, RPS)])
    pltpu.sync_copy(ones_hbm, ones_v)
    pltpu.sync_copy(dst_hbm.at[wid], dst_v)
    plsc.subcore_barrier()

    @pl.loop(0, NBH)
    def _(j):
        pltpu.sync_copy(ones_v, acc.at[dst_v.at[j]], add=True)

    plsc.subcore_barrier()
    pltpu.sync_copy(acc.at[pl.ds(s * RPS, RPS)],
                    out_hbm.at[pl.ds(s * RPS, RPS), pl.ds(c * 16, 16)])


_scatter64 = _make_edge_scatter(64)
_scatter32 = _make_edge_scatter(32)


def _pre_body(x_ref, w_ref, degp_ref, q_ref, dinv_ref):
    deg = degp_ref[:N, 0:1] + degp_ref[:N, 16:17] + 1.0
    dinv = lax.rsqrt(deg)
    dinv_ref[...] = dinv
    q_ref[...] = jnp.dot(x_ref[...], w_ref[...],
                         preferred_element_type=jnp.float32) * dinv


def _mid_body(sp_ref, q_ref, dinv_ref, b_ref, g_ref, be_ref, w_ref, qn_ref):
    hw = w_ref.shape[1]
    dinv = dinv_ref[...]
    t = (sp_ref[:N, :] + q_ref[...]) * dinv + b_ref[...]
    m = jnp.mean(t, axis=0, keepdims=True)
    cdev = t - m
    v = jnp.mean(cdev * cdev, axis=0, keepdims=True)
    h = jnp.maximum(cdev * lax.rsqrt(v + 1e-5) * g_ref[...] + be_ref[...], 0.0)
    qn = jnp.dot(h, w_ref[...], preferred_element_type=jnp.float32) * dinv
    qn_ref[:, :hw] = qn
    if hw < 128:
        qn_ref[:, hw:] = jnp.zeros((N, 128 - hw), jnp.float32)


def _post_body(sp_ref, q_ref, dinv_ref, b_ref, g_ref, be_ref, batch_ref,
               fcw_ref, fcb_ref, out_ref):
    t = (sp_ref[:N, :64] + q_ref[:, :64]) * dinv_ref[...] + b_ref[...]
    m = jnp.mean(t, axis=0, keepdims=True)
    cdev = t - m
    v = jnp.mean(cdev * cdev, axis=0, keepdims=True)
    h = jnp.maximum(cdev * lax.rsqrt(v + 1e-5) * g_ref[...] + be_ref[...], 0.0)
    onehot_t = (batch_ref[...] ==
                lax.broadcasted_iota(jnp.int32, (G, N), 0)).astype(jnp.float32)
    sums = jnp.dot(onehot_t, h, preferred_element_type=jnp.float32)
    cnt = jnp.sum(onehot_t, axis=1, keepdims=True)
    pooled = sums / jnp.maximum(cnt, 1.0)
    out_ref[...] = jnp.dot(pooled, fcw_ref[...],
                           preferred_element_type=jnp.float32) + fcb_ref[...]


def kernel(x, edge_index, batch, W1, b1, g1, be1, W2, b2, g2, be2,
           W3, b3, g3, be3, fcW, fcb):
    f32 = jnp.float32
    pad = EPAD - E
    srcp = jnp.concatenate(
        [edge_index[0], jnp.zeros((pad,), jnp.int32)]).reshape(16, NBT, K)
    dstp = jnp.concatenate(
        [edge_index[1], jnp.full((pad,), N, jnp.int32)]).reshape(16, NBT, K)
    dstp32 = dstp.reshape(32, NBH, K)
    z128 = jnp.zeros((RPS, 128), f32)
    ones16 = jnp.ones((K, 16), f32)

    degp = _deg_kernel(dstp32, ones16, z128)

    q1, dinv = pl.pallas_call(
        _pre_body,
        out_shape=[jax.ShapeDtypeStruct((N, 128), f32),
                   jax.ShapeDtypeStruct((N, 1), f32)],
    )(x, W1, degp)

    s1 = _scatter64(q1, srcp, dstp, z128)
    q2 = pl.pallas_call(
        _mid_body, out_shape=jax.ShapeDtypeStruct((N, 128), f32),
    )(s1, q1, dinv, b1.reshape(1, -1), g1.reshape(1, -1), be1.reshape(1, -1),
      W2)

    s2 = _scatter64(q2, srcp, dstp, z128)
    q3 = pl.pallas_call(
        _mid_body, out_shape=jax.ShapeDtypeStruct((N, 128), f32),
    )(s2, q2, dinv, b2.reshape(1, -1), g2.reshape(1, -1), be2.reshape(1, -1),
      W3)

    s3 = _scatter32(q3, srcp, dstp, z128)
    out = pl.pallas_call(
        _post_body, out_shape=jax.ShapeDtypeStruct((G, 10), f32),
    )(s3, q3, dinv, b3.reshape(1, -1), g3.reshape(1, -1), be3.reshape(1, -1),
      batch.reshape(1, -1), fcW, fcb.reshape(1, -1))
    return out


# speedup vs baseline: 26.5193x; 1.0335x over previous
import functools

import jax
import jax.numpy as jnp
from jax import lax
from jax.experimental import pallas as pl
from jax.experimental.pallas import tpu as pltpu
from jax.experimental.pallas import tpu_sc as plsc

N = 10000
E = 320000
G = 128

K = 80
NBT = E // (16 * K)
NBD = E // (32 * K)
NPAD = 10016
RPS = NPAD // 16
TRS = N // 16


_MESH = plsc.VectorSubcoreMesh(core_axis_name="c", subcore_axis_name="s")
_SC_PARAMS = pltpu.CompilerParams(use_tc_tiling_on_sc=False)


def _make_edge_scatter(hh):
    @functools.partial(
        pl.kernel,
        mesh=_MESH,
        compiler_params=_SC_PARAMS,
        out_type=jax.ShapeDtypeStruct((NPAD, 128), jnp.float32),
        scratch_types=[
            pltpu.VMEM((NBT, K), jnp.int32),
            pltpu.VMEM((NBT, K), jnp.int32),
            pltpu.VMEM((K, hh), jnp.float32),
            pltpu.VMEM((K, hh), jnp.float32),
            pltpu.VMEM_SHARED((N, hh), jnp.float32),
            pltpu.VMEM_SHARED((NPAD, hh), jnp.float32),
            pltpu.SemaphoreType.DMA,
            pltpu.SemaphoreType.DMA,
            pltpu.SemaphoreType.DMA,
            pltpu.SemaphoreType.DMA,
        ],
    )
    def scatter_kernel(q_hbm, src_hbm, dst_hbm, z_hbm, out_hbm,
                       src_v, dst_v, b0, b1, table, acc, g0, g1, s0, s1):
        c = lax.axis_index("c")
        s = lax.axis_index("s")
        bufs = [b0, b1]
        gsems = [g0, g1]
        ssems = [s0, s1]
        pltpu.sync_copy(z_hbm.at[:, pl.ds(0, hh)],
                        acc.at[pl.ds(s * RPS, RPS)])
        pltpu.sync_copy(q_hbm.at[pl.ds(s * TRS, TRS), pl.ds(c * hh, hh)],
                        table.at[pl.ds(s * TRS, TRS)])
        pltpu.sync_copy(src_hbm.at[s], src_v)
        pltpu.sync_copy(dst_hbm.at[s], dst_v)
        plsc.subcore_barrier()

        def g_issue(j, b):
            pltpu.async_copy(table.at[src_v.at[j]], bufs[b], gsems[b])

        def g_wait(j, b):
            pltpu.make_async_copy(
                table.at[src_v.at[j]], bufs[b], gsems[b]).wait()

        def s_issue(j, b):
            pltpu.async_copy(bufs[b], acc.at[dst_v.at[j]], ssems[b],
                             add=True)

        def s_wait(j, b):
            pltpu.make_async_copy(
                bufs[b], acc.at[dst_v.at[j]], ssems[b]).wait()

        g_issue(0, 0)
        g_wait(0, 0); s_issue(0, 0); g_issue(1, 1)
        g_wait(1, 1); s_issue(1, 1); s_wait(0, 0); g_issue(2, 0)

        @pl.loop(2, NBT - 2, step=2)
        def _(jr):
            for b in range(2):
                j = jr + b
                g_wait(j, b)
                s_issue(j, b)
                s_wait(j - 1, 1 - b)
                g_issue(j + 1, 1 - b)

        jr = NBT - 2
        g_wait(jr, 0); s_issue(jr, 0); s_wait(jr - 1, 1)
        g_issue(jr + 1, 1)
        g_wait(jr + 1, 1); s_issue(jr + 1, 1); s_wait(jr, 0)
        s_wait(jr + 1, 1)

        plsc.subcore_barrier()
        pltpu.sync_copy(acc.at[pl.ds(s * RPS, RPS)],
                        out_hbm.at[pl.ds(s * RPS, RPS), pl.ds(c * hh, hh)])

    return scatter_kernel


@functools.partial(
    pl.kernel,
    mesh=_MESH,
    compiler_params=_SC_PARAMS,
    out_type=jax.ShapeDtypeStruct((NPAD, 128), jnp.float32),
    scratch_types=[
        pltpu.VMEM((NBD, K), jnp.int32),
        pltpu.VMEM((K, 16), jnp.float32),
        pltpu.VMEM_SHARED((NPAD, 16), jnp.float32),
        pltpu.SemaphoreType.DMA,
    ],
)
def _deg_kernel(dst_hbm, ones_hbm, z_hbm, out_hbm, dst_v, ones_v, acc, sem):
    c = lax.axis_index("c")
    s = lax.axis_index("s")
    wid = c * 16 + s
    pltpu.sync_copy(z_hbm.at[:, pl.ds(0, 16)], acc.at[pl.ds(s * RPS, RPS)])
    pltpu.sync_copy(ones_hbm, ones_v)
    pltpu.sync_copy(dst_hbm.at[wid], dst_v)
    plsc.subcore_barrier()

    @pl.loop(0, NBD, step=5)
    def _(j):
        for i in range(5):
            pltpu.async_copy(ones_v, acc.at[dst_v.at[j + i]], sem, add=True)
        for i in range(5):
            pltpu.make_async_copy(
                ones_v, acc.at[dst_v.at[j + i]], sem).wait()

    plsc.subcore_barrier()
    pltpu.sync_copy(acc.at[pl.ds(s * RPS, RPS)],
                    out_hbm.at[pl.ds(s * RPS, RPS), pl.ds(c * 16, 16)])


_scatter64 = _make_edge_scatter(64)
_scatter32 = _make_edge_scatter(32)


def _pre_body(x_ref, w_ref, degp_ref, q_ref, dinv_ref):
    deg = degp_ref[:N, 0:1] + degp_ref[:N, 16:17] + 1.0
    dinv = lax.rsqrt(deg)
    dinv_ref[...] = dinv
    q_ref[...] = jnp.dot(x_ref[...], w_ref[...],
                         preferred_element_type=jnp.float32) * dinv


def _mid_body(sp_ref, q_ref, dinv_ref, b_ref, g_ref, be_ref, w_ref, qn_ref):
    hw = w_ref.shape[1]
    dinv = dinv_ref[...]
    t = (sp_ref[:N, :] + q_ref[...]) * dinv + b_ref[...]
    m = jnp.mean(t, axis=0, keepdims=True)
    cdev = t - m
    v = jnp.mean(cdev * cdev, axis=0, keepdims=True)
    h = jnp.maximum(cdev * lax.rsqrt(v + 1e-5) * g_ref[...] + be_ref[...], 0.0)
    qn = jnp.dot(h, w_ref[...], preferred_element_type=jnp.float32) * dinv
    qn_ref[:, :hw] = qn
    if hw < 128:
        qn_ref[:, hw:] = jnp.zeros((N, 128 - hw), jnp.float32)


def _post_body(sp_ref, q_ref, dinv_ref, b_ref, g_ref, be_ref, batch_ref,
               fcw_ref, fcb_ref, out_ref):
    t = (sp_ref[:N, :64] + q_ref[:, :64]) * dinv_ref[...] + b_ref[...]
    m = jnp.mean(t, axis=0, keepdims=True)
    cdev = t - m
    v = jnp.mean(cdev * cdev, axis=0, keepdims=True)
    h = jnp.maximum(cdev * lax.rsqrt(v + 1e-5) * g_ref[...] + be_ref[...], 0.0)
    onehot_t = (batch_ref[...] ==
                lax.broadcasted_iota(jnp.int32, (G, N), 0)).astype(jnp.float32)
    sums = jnp.dot(onehot_t, h, preferred_element_type=jnp.float32)
    cnt = jnp.sum(onehot_t, axis=1, keepdims=True)
    pooled = sums / jnp.maximum(cnt, 1.0)
    out_ref[...] = jnp.dot(pooled, fcw_ref[...],
                           preferred_element_type=jnp.float32) + fcb_ref[...]


def kernel(x, edge_index, batch, W1, b1, g1, be1, W2, b2, g2, be2,
           W3, b3, g3, be3, fcW, fcb):
    f32 = jnp.float32
    srcp = edge_index[0].reshape(16, NBT, K)
    dstp = edge_index[1].reshape(16, NBT, K)
    dstp32 = dstp.reshape(32, NBD, K)
    z128 = jnp.zeros((RPS, 128), f32)
    ones16 = jnp.ones((K, 16), f32)

    degp = _deg_kernel(dstp32, ones16, z128)

    q1, dinv = pl.pallas_call(
        _pre_body,
        out_shape=[jax.ShapeDtypeStruct((N, 128), f32),
                   jax.ShapeDtypeStruct((N, 1), f32)],
    )(x, W1, degp)

    s1 = _scatter64(q1, srcp, dstp, z128)
    q2 = pl.pallas_call(
        _mid_body, out_shape=jax.ShapeDtypeStruct((N, 128), f32),
    )(s1, q1, dinv, b1.reshape(1, -1), g1.reshape(1, -1), be1.reshape(1, -1),
      W2)

    s2 = _scatter64(q2, srcp, dstp, z128)
    q3 = pl.pallas_call(
        _mid_body, out_shape=jax.ShapeDtypeStruct((N, 128), f32),
    )(s2, q2, dinv, b2.reshape(1, -1), g2.reshape(1, -1), be2.reshape(1, -1),
      W3)

    s3 = _scatter32(q3, srcp, dstp, z128)
    out = pl.pallas_call(
        _post_body, out_shape=jax.ShapeDtypeStruct((G, 10), f32),
    )(s3, q3, dinv, b3.reshape(1, -1), g3.reshape(1, -1), be3.reshape(1, -1),
      batch.reshape(1, -1), fcW, fcb.reshape(1, -1))
    return out


# speedup vs baseline: 27.1780x; 1.0248x over previous
import functools

import jax
import jax.numpy as jnp
from jax import lax
from jax.experimental import pallas as pl
from jax.experimental.pallas import tpu as pltpu
from jax.experimental.pallas import tpu_sc as plsc

N = 10000
E = 320000
G = 128

K = 80
NBT = E // (16 * K)
NBD = E // (32 * K)
NPAD = 10016
RPS = NPAD // 16
TRS = N // 16


_MESH = plsc.VectorSubcoreMesh(core_axis_name="c", subcore_axis_name="s")
_SC_PARAMS = pltpu.CompilerParams(use_tc_tiling_on_sc=False)


def _make_edge_scatter(hh):
    @functools.partial(
        pl.kernel,
        mesh=_MESH,
        compiler_params=_SC_PARAMS,
        out_type=jax.ShapeDtypeStruct((NPAD, 128), jnp.float32),
        scratch_types=[
            pltpu.VMEM((NBT, K), jnp.int32),
            pltpu.VMEM((NBT, K), jnp.int32),
            pltpu.VMEM((K, hh), jnp.float32),
            pltpu.VMEM((K, hh), jnp.float32),
            pltpu.VMEM_SHARED((N, hh), jnp.float32),
            pltpu.VMEM_SHARED((NPAD, hh), jnp.float32),
            pltpu.SemaphoreType.DMA,
            pltpu.SemaphoreType.DMA,
            pltpu.SemaphoreType.DMA,
            pltpu.SemaphoreType.DMA,
        ],
    )
    def scatter_kernel(q_hbm, edges_hbm, z_hbm, out_hbm,
                       src_v, dst_v, b0, b1, table, acc, g0, g1, s0, s1):
        c = lax.axis_index("c")
        s = lax.axis_index("s")
        bufs = [b0, b1]
        gsems = [g0, g1]
        ssems = [s0, s1]
        pltpu.sync_copy(z_hbm.at[:, pl.ds(0, hh)],
                        acc.at[pl.ds(s * RPS, RPS)])
        pltpu.sync_copy(q_hbm.at[pl.ds(s * TRS, TRS), pl.ds(c * hh, hh)],
                        table.at[pl.ds(s * TRS, TRS)])
        pltpu.sync_copy(edges_hbm.at[0, s], src_v)
        pltpu.sync_copy(edges_hbm.at[1, s], dst_v)
        plsc.subcore_barrier()

        def g_issue(j, b):
            pltpu.async_copy(table.at[src_v.at[j]], bufs[b], gsems[b])

        def g_wait(j, b):
            pltpu.make_async_copy(
                table.at[src_v.at[j]], bufs[b], gsems[b]).wait()

        def s_issue(j, b):
            pltpu.async_copy(bufs[b], acc.at[dst_v.at[j]], ssems[b],
                             add=True)

        def s_wait(j, b):
            pltpu.make_async_copy(
                bufs[b], acc.at[dst_v.at[j]], ssems[b]).wait()

        g_issue(0, 0)
        g_wait(0, 0); s_issue(0, 0); g_issue(1, 1)
        g_wait(1, 1); s_issue(1, 1); s_wait(0, 0); g_issue(2, 0)

        @pl.loop(2, NBT - 2, step=2)
        def _(jr):
            for b in range(2):
                j = jr + b
                g_wait(j, b)
                s_issue(j, b)
                s_wait(j - 1, 1 - b)
                g_issue(j + 1, 1 - b)

        jr = NBT - 2
        g_wait(jr, 0); s_issue(jr, 0); s_wait(jr - 1, 1)
        g_issue(jr + 1, 1)
        g_wait(jr + 1, 1); s_issue(jr + 1, 1); s_wait(jr, 0)
        s_wait(jr + 1, 1)

        plsc.subcore_barrier()
        pltpu.sync_copy(acc.at[pl.ds(s * RPS, RPS)],
                        out_hbm.at[pl.ds(s * RPS, RPS), pl.ds(c * hh, hh)])

    return scatter_kernel


@functools.partial(
    pl.kernel,
    mesh=_MESH,
    compiler_params=_SC_PARAMS,
    out_type=jax.ShapeDtypeStruct((NPAD, 128), jnp.float32),
    scratch_types=[
        pltpu.VMEM((NBD, K), jnp.int32),
        pltpu.VMEM((K, 16), jnp.float32),
        pltpu.VMEM_SHARED((NPAD, 16), jnp.float32),
        pltpu.SemaphoreType.DMA,
    ],
)
def _deg_kernel(edges_hbm, ones_hbm, z_hbm, out_hbm, dst_v, ones_v, acc, sem):
    c = lax.axis_index("c")
    s = lax.axis_index("s")
    pltpu.sync_copy(z_hbm.at[:, pl.ds(0, 16)], acc.at[pl.ds(s * RPS, RPS)])
    pltpu.sync_copy(ones_hbm, ones_v)
    pltpu.sync_copy(edges_hbm.at[1, s, pl.ds(c * NBD, NBD)], dst_v)
    plsc.subcore_barrier()

    @pl.loop(0, NBD, step=5)
    def _(j):
        for i in range(5):
            pltpu.async_copy(ones_v, acc.at[dst_v.at[j + i]], sem, add=True)
        for i in range(5):
            pltpu.make_async_copy(
                ones_v, acc.at[dst_v.at[j + i]], sem).wait()

    plsc.subcore_barrier()
    pltpu.sync_copy(acc.at[pl.ds(s * RPS, RPS)],
                    out_hbm.at[pl.ds(s * RPS, RPS), pl.ds(c * 16, 16)])


_scatter64 = _make_edge_scatter(64)
_scatter32 = _make_edge_scatter(32)


def _pre_body(x_ref, w_ref, degp_ref, q_ref, dinv_ref):
    deg = degp_ref[:N, 0:1] + degp_ref[:N, 16:17] + 1.0
    dinv = lax.rsqrt(deg)
    dinv_ref[...] = dinv
    q_ref[...] = jnp.dot(x_ref[...], w_ref[...],
                         preferred_element_type=jnp.float32) * dinv


def _mid_body(sp_ref, q_ref, dinv_ref, b_ref, g_ref, be_ref, w_ref, qn_ref):
    hw = w_ref.shape[1]
    dinv = dinv_ref[...]
    t = (sp_ref[:N, :] + q_ref[...]) * dinv + b_ref[...]
    m = jnp.mean(t, axis=0, keepdims=True)
    cdev = t - m
    v = jnp.mean(cdev * cdev, axis=0, keepdims=True)
    h = jnp.maximum(cdev * lax.rsqrt(v + 1e-5) * g_ref[...] + be_ref[...], 0.0)
    qn = jnp.dot(h, w_ref[...], preferred_element_type=jnp.float32) * dinv
    qn_ref[:, :hw] = qn
    if hw < 128:
        qn_ref[:, hw:] = jnp.zeros((N, 128 - hw), jnp.float32)


def _post_body(sp_ref, q_ref, dinv_ref, b_ref, g_ref, be_ref, batch_ref,
               fcw_ref, fcb_ref, out_ref):
    t = (sp_ref[:N, :64] + q_ref[:, :64]) * dinv_ref[...] + b_ref[...]
    m = jnp.mean(t, axis=0, keepdims=True)
    cdev = t - m
    v = jnp.mean(cdev * cdev, axis=0, keepdims=True)
    h = jnp.maximum(cdev * lax.rsqrt(v + 1e-5) * g_ref[...] + be_ref[...], 0.0)
    onehot_t = (batch_ref[...] ==
                lax.broadcasted_iota(jnp.int32, (G, N), 0)).astype(jnp.float32)
    sums = jnp.dot(onehot_t, h, preferred_element_type=jnp.float32)
    cnt = jnp.sum(onehot_t, axis=1, keepdims=True)
    pooled = sums / jnp.maximum(cnt, 1.0)
    out_ref[...] = jnp.dot(pooled, fcw_ref[...],
                           preferred_element_type=jnp.float32) + fcb_ref[...]


def kernel(x, edge_index, batch, W1, b1, g1, be1, W2, b2, g2, be2,
           W3, b3, g3, be3, fcW, fcb):
    f32 = jnp.float32
    edges4 = edge_index.reshape(2, 16, NBT, K)
    z128 = jnp.zeros((RPS, 128), f32)
    ones16 = jnp.ones((K, 16), f32)

    degp = _deg_kernel(edges4, ones16, z128)

    q1, dinv = pl.pallas_call(
        _pre_body,
        out_shape=[jax.ShapeDtypeStruct((N, 128), f32),
                   jax.ShapeDtypeStruct((N, 1), f32)],
    )(x, W1, degp)

    s1 = _scatter64(q1, edges4, z128)
    q2 = pl.pallas_call(
        _mid_body, out_shape=jax.ShapeDtypeStruct((N, 128), f32),
    )(s1, q1, dinv, b1.reshape(1, -1), g1.reshape(1, -1), be1.reshape(1, -1),
      W2)

    s2 = _scatter64(q2, edges4, z128)
    q3 = pl.pallas_call(
        _mid_body, out_shape=jax.ShapeDtypeStruct((N, 128), f32),
    )(s2, q2, dinv, b2.reshape(1, -1), g2.reshape(1, -1), be2.reshape(1, -1),
      W3)

    s3 = _scatter32(q3, edges4, z128)
    out = pl.pallas_call(
        _post_body, out_shape=jax.ShapeDtypeStruct((G, 10), f32),
    )(s3, q3, dinv, b3.reshape(1, -1), g3.reshape(1, -1), be3.reshape(1, -1),
      batch.reshape(1, -1), fcW, fcb.reshape(1, -1))
    return out
